# Initial kernel scaffold; baseline (speedup 1.0000x reference)
import jax
import jax.numpy as jnp
from jax.experimental import pallas as pl


def kernel(feat, bus_type, bus_feature_dims, W, bias):
    raise NotImplementedError("write your pallas kernel here")


# speedup vs baseline: 2.2349x; 2.2349x over previous
import functools

import jax
import jax.numpy as jnp
from jax import lax
from jax.experimental import pallas as pl
from jax.experimental.pallas import tpu as pltpu
from jax.experimental.pallas import tpu_sc as plsc

N = 16384
F = 128
E = 8
D = 4096

R = 256
NPAD = N + E * R
NB = NPAD // R
NW = 32
PW = NPAD // NW
GCH = 96
NG = PW // GCH
SCH = 24
NSC = PW // SCH

_MESH = dict(core_axis_name="c", subcore_axis_name="s")


def _sc_gather(feat, perm3):
    @functools.partial(
        pl.kernel,
        mesh=plsc.VectorSubcoreMesh(**_MESH),
        out_type=jax.ShapeDtypeStruct((NPAD, F), jnp.float32),
        scratch_types=[
            pltpu.VMEM((NG, GCH), jnp.int32),
            pltpu.VMEM((PW, F), jnp.float32),
            pltpu.SemaphoreType.DMA,
        ],
    )
    def gk(feat_hbm, idx_hbm, out_hbm, idx_v, rows_v, sem):
        wid = lax.axis_index("s") * 2 + lax.axis_index("c")
        pltpu.sync_copy(idx_hbm.at[wid], idx_v)
        copies = [
            pltpu.async_copy(
                feat_hbm.at[idx_v.at[k]], rows_v.at[pl.ds(k * GCH, GCH)], sem
            )
            for k in range(NG)
        ]
        for c in copies:
            c.wait()
        pltpu.sync_copy(rows_v, out_hbm.at[pl.ds(wid * PW, PW)])

    return gk(feat, perm3)


def _sc_scatter(y, perm3):
    @functools.partial(
        pl.kernel,
        mesh=plsc.VectorSubcoreMesh(**_MESH),
        out_type=jax.ShapeDtypeStruct((N, D), jnp.float32),
        scratch_types=[
            pltpu.VMEM((NSC, SCH), jnp.int32),
            pltpu.VMEM((SCH, D), jnp.float32),
            pltpu.SemaphoreType.DMA,
        ],
    )
    def sk(y_hbm, idx_hbm, out_hbm, idx_v, buf_v, sem):
        wid = lax.axis_index("s") * 2 + lax.axis_index("c")
        base = wid * PW
        pltpu.sync_copy(idx_hbm.at[wid], idx_v)
        for c in range(NSC):
            pltpu.sync_copy(y_hbm.at[pl.ds(base + c * SCH, SCH)], buf_v)
            pltpu.async_copy(buf_v, out_hbm.at[idx_v.at[c]], sem).wait()

    return sk(y, perm3)


def _tc_matmul(x, W, bias, block_expert, bfd):
    def body(be_ref, bfd_ref, x_ref, w_ref, b_ref, o_ref):
        i = pl.program_id(0)
        end = jnp.minimum(bfd_ref[be_ref[i]], F)
        colmask = (lax.broadcasted_iota(jnp.int32, (1, F), 1) < end).astype(
            jnp.float32
        )
        xm = x_ref[...] * colmask
        acc = jnp.dot(xm, w_ref[0], preferred_element_type=jnp.float32)
        o_ref[...] = jnp.tanh(acc + b_ref[0])

    grid_spec = pltpu.PrefetchScalarGridSpec(
        num_scalar_prefetch=2,
        grid=(NB,),
        in_specs=[
            pl.BlockSpec((R, F), lambda i, be, bfd: (i, 0)),
            pl.BlockSpec((1, F, D), lambda i, be, bfd: (be[i], 0, 0)),
            pl.BlockSpec((1, 1, D), lambda i, be, bfd: (be[i], 0, 0)),
        ],
        out_specs=pl.BlockSpec((R, D), lambda i, be, bfd: (i, 0)),
    )
    return pl.pallas_call(
        body,
        grid_spec=grid_spec,
        out_shape=jax.ShapeDtypeStruct((NPAD, D), jnp.float32),
        compiler_params=pltpu.CompilerParams(dimension_semantics=("arbitrary",)),
    )(block_expert, bfd, x, W, bias.reshape(E, 1, D))


def _route(bus_type):
    bt = bus_type.astype(jnp.int32)
    oh = bt[:, None] == jnp.arange(E, dtype=jnp.int32)[None, :]
    counts = jnp.sum(oh.astype(jnp.int32), axis=0)
    order = jnp.argsort(bt).astype(jnp.int32)
    cum_excl = jnp.cumsum(counts) - counts
    padded = ((counts + R - 1) // R) * R
    ends_p = jnp.cumsum(padded)
    starts_p = ends_p - padded
    slot = jnp.arange(NPAD, dtype=jnp.int32)
    grp = jnp.searchsorted(ends_p, slot, side="right").astype(jnp.int32)
    g = jnp.minimum(grp, E - 1)
    off = slot - starts_p[g]
    src = jnp.clip(cum_excl[g] + jnp.minimum(off, counts[g] - 1), 0, N - 1)
    perm = order[src]
    block_expert = bt[perm[::R]]
    return perm, block_expert


def kernel(feat, bus_type, bus_feature_dims, W, bias):
    perm, block_expert = _route(bus_type)
    gathered = _sc_gather(feat, perm.reshape(NW, NG, GCH))
    y = _tc_matmul(
        gathered, W, bias, block_expert, bus_feature_dims.astype(jnp.int32)
    )
    return _sc_scatter(y, perm.reshape(NW, NSC, SCH))


# speedup vs baseline: 2.6426x; 1.1824x over previous
import functools

import jax
import jax.numpy as jnp
from jax import lax
from jax.experimental import pallas as pl
from jax.experimental.pallas import tpu as pltpu
from jax.experimental.pallas import tpu_sc as plsc

N = 16384
F = 128
E = 8
D = 4096

R = 256
NPAD = N + E * R
NB = NPAD // R
NW = 32
PW = NPAD // NW
GCH = 96
NG = PW // GCH
PW2 = N // NW
SCH = 8
NSC = PW2 // SCH

_MESH = dict(core_axis_name="c", subcore_axis_name="s")


def _sc_gather(feat, perm3):
    @functools.partial(
        pl.kernel,
        mesh=plsc.VectorSubcoreMesh(**_MESH),
        out_type=jax.ShapeDtypeStruct((NPAD, F), jnp.float32),
        scratch_types=[
            pltpu.VMEM((NG, GCH), jnp.int32),
            pltpu.VMEM((PW, F), jnp.float32),
            pltpu.SemaphoreType.DMA,
        ],
    )
    def gk(feat_hbm, idx_hbm, out_hbm, idx_v, rows_v, sem):
        wid = lax.axis_index("s") * 2 + lax.axis_index("c")
        pltpu.sync_copy(idx_hbm.at[wid], idx_v)
        copies = [
            pltpu.async_copy(
                feat_hbm.at[idx_v.at[k]], rows_v.at[pl.ds(k * GCH, GCH)], sem
            )
            for k in range(NG)
        ]
        for c in copies:
            c.wait()
        pltpu.sync_copy(rows_v, out_hbm.at[pl.ds(wid * PW, PW)])

    return gk(feat, perm3)


def _sc_unpermute(y, pos3):
    @functools.partial(
        pl.kernel,
        mesh=plsc.VectorSubcoreMesh(**_MESH),
        out_type=jax.ShapeDtypeStruct((N, D), jnp.float32),
        scratch_types=[
            pltpu.VMEM((NSC, SCH), jnp.int32),
            pltpu.VMEM((2, SCH, D), jnp.float32),
            pltpu.SemaphoreType.DMA,
            pltpu.SemaphoreType.DMA,
        ],
    )
    def uk(y_hbm, idx_hbm, out_hbm, idx_v, buf_v, sem_in, sem_out):
        wid = lax.axis_index("s") * 2 + lax.axis_index("c")
        base = wid * PW2
        pltpu.sync_copy(idx_hbm.at[wid], idx_v)
        reads = [None] * NSC
        writes = [None] * NSC
        reads[0] = pltpu.async_copy(y_hbm.at[idx_v.at[0]], buf_v.at[0], sem_in)
        reads[1] = pltpu.async_copy(y_hbm.at[idx_v.at[1]], buf_v.at[1], sem_in)
        for c in range(NSC):
            b = c % 2
            reads[c].wait()
            writes[c] = pltpu.async_copy(
                buf_v.at[b], out_hbm.at[pl.ds(base + c * SCH, SCH)], sem_out
            )
            if c + 2 < NSC:
                writes[c].wait()
                reads[c + 2] = pltpu.async_copy(
                    y_hbm.at[idx_v.at[c + 2]], buf_v.at[b], sem_in
                )
        writes[NSC - 2].wait()
        writes[NSC - 1].wait()

    return uk(y, pos3)


def _tc_matmul(x, W, bias, block_expert, bfd):
    def body(be_ref, bfd_ref, x_ref, w_ref, b_ref, o_ref):
        i = pl.program_id(0)
        end = jnp.minimum(bfd_ref[be_ref[i]], F)
        colmask = (lax.broadcasted_iota(jnp.int32, (1, F), 1) < end).astype(
            jnp.float32
        )
        xm = x_ref[...] * colmask
        acc = jnp.dot(xm, w_ref[0], preferred_element_type=jnp.float32)
        o_ref[...] = jnp.tanh(acc + b_ref[0])

    grid_spec = pltpu.PrefetchScalarGridSpec(
        num_scalar_prefetch=2,
        grid=(NB,),
        in_specs=[
            pl.BlockSpec((R, F), lambda i, be, bfd: (i, 0)),
            pl.BlockSpec((1, F, D), lambda i, be, bfd: (be[i], 0, 0)),
            pl.BlockSpec((1, 1, D), lambda i, be, bfd: (be[i], 0, 0)),
        ],
        out_specs=pl.BlockSpec((R, D), lambda i, be, bfd: (i, 0)),
    )
    return pl.pallas_call(
        body,
        grid_spec=grid_spec,
        out_shape=jax.ShapeDtypeStruct((NPAD, D), jnp.float32),
        compiler_params=pltpu.CompilerParams(dimension_semantics=("arbitrary",)),
    )(block_expert, bfd, x, W, bias.reshape(E, 1, D))


def _route(bus_type):
    bt = bus_type.astype(jnp.int32)
    ohi = (bt[:, None] == jnp.arange(E, dtype=jnp.int32)[None, :]).astype(
        jnp.int32
    )
    counts = jnp.sum(ohi, axis=0)
    order = jnp.argsort(bt).astype(jnp.int32)
    cum_excl = jnp.cumsum(counts) - counts
    padded = ((counts + R - 1) // R) * R
    ends_p = jnp.cumsum(padded)
    starts_p = ends_p - padded
    slot = jnp.arange(NPAD, dtype=jnp.int32)
    grp = jnp.searchsorted(ends_p, slot, side="right").astype(jnp.int32)
    g = jnp.minimum(grp, E - 1)
    off = slot - starts_p[g]
    src = jnp.clip(cum_excl[g] + jnp.minimum(off, counts[g] - 1), 0, N - 1)
    perm = order[src]
    block_expert = bt[perm[::R]]
    rank = jnp.take_along_axis(
        jnp.cumsum(ohi, axis=0) - ohi, bt[:, None], axis=1
    )[:, 0]
    pos = starts_p[bt] + rank
    return perm, pos, block_expert


def kernel(feat, bus_type, bus_feature_dims, W, bias):
    perm, pos, block_expert = _route(bus_type)
    gathered = _sc_gather(feat, perm.reshape(NW, NG, GCH))
    y = _tc_matmul(
        gathered, W, bias, block_expert, bus_feature_dims.astype(jnp.int32)
    )
    return _sc_unpermute(y, pos.reshape(NW, NSC, SCH))


# speedup vs baseline: 2.6460x; 1.0013x over previous
import functools

import jax
import jax.numpy as jnp
from jax import lax
from jax.experimental import pallas as pl
from jax.experimental.pallas import tpu as pltpu
from jax.experimental.pallas import tpu_sc as plsc

N = 16384
F = 128
E = 8
D = 4096

R = 256
NPAD = N + E * R
NB = NPAD // R
NW = 32
PW = NPAD // NW
GCH = 96
NG = PW // GCH
PW2 = N // NW
SCH = 8
NSC = PW2 // SCH

_MESH = dict(core_axis_name="c", subcore_axis_name="s")


def _sc_gather(feat, perm3):
    @functools.partial(
        pl.kernel,
        mesh=plsc.VectorSubcoreMesh(**_MESH),
        out_type=jax.ShapeDtypeStruct((NPAD, F), jnp.float32),
        scratch_types=[
            pltpu.VMEM((NG, GCH), jnp.int32),
            pltpu.VMEM((PW, F), jnp.float32),
            pltpu.SemaphoreType.DMA,
        ],
    )
    def gk(feat_hbm, idx_hbm, out_hbm, idx_v, rows_v, sem):
        wid = lax.axis_index("s") * 2 + lax.axis_index("c")
        pltpu.sync_copy(idx_hbm.at[wid], idx_v)
        copies = [
            pltpu.async_copy(
                feat_hbm.at[idx_v.at[k]], rows_v.at[pl.ds(k * GCH, GCH)], sem
            )
            for k in range(NG)
        ]
        for c in copies:
            c.wait()
        pltpu.sync_copy(rows_v, out_hbm.at[pl.ds(wid * PW, PW)])

    return gk(feat, perm3)


def _sc_unpermute(y, pos3):
    @functools.partial(
        pl.kernel,
        mesh=plsc.VectorSubcoreMesh(**_MESH),
        out_type=jax.ShapeDtypeStruct((N, D), jnp.float32),
        scratch_types=[
            pltpu.VMEM((NSC, SCH), jnp.int32),
            pltpu.VMEM((2, SCH, D), jnp.float32),
            pltpu.SemaphoreType.DMA,
            pltpu.SemaphoreType.DMA,
        ],
    )
    def uk(y_hbm, idx_hbm, out_hbm, idx_v, buf_v, sem_in, sem_out):
        wid = lax.axis_index("s") * 2 + lax.axis_index("c")
        base = wid * PW2
        pltpu.sync_copy(idx_hbm.at[wid], idx_v)
        reads = [None] * NSC
        writes = [None] * NSC
        reads[0] = pltpu.async_copy(y_hbm.at[idx_v.at[0]], buf_v.at[0], sem_in)
        reads[1] = pltpu.async_copy(y_hbm.at[idx_v.at[1]], buf_v.at[1], sem_in)
        for c in range(NSC):
            b = c % 2
            reads[c].wait()
            writes[c] = pltpu.async_copy(
                buf_v.at[b], out_hbm.at[pl.ds(base + c * SCH, SCH)], sem_out
            )
            if c + 2 < NSC:
                writes[c].wait()
                reads[c + 2] = pltpu.async_copy(
                    y_hbm.at[idx_v.at[c + 2]], buf_v.at[b], sem_in
                )
        writes[NSC - 2].wait()
        writes[NSC - 1].wait()

    return uk(y, pos3)


def _tc_matmul(x, W, bias, block_expert, bfd):
    def body(be_ref, bfd_ref, x_ref, w_ref, b_ref, o_ref):
        i = pl.program_id(0)
        end = jnp.minimum(bfd_ref[be_ref[i]], F)
        colmask = (lax.broadcasted_iota(jnp.int32, (1, F), 1) < end).astype(
            jnp.float32
        )
        xm = (x_ref[...] * colmask).astype(jnp.bfloat16)
        acc = jnp.dot(xm, w_ref[0], preferred_element_type=jnp.float32)
        o_ref[...] = jnp.tanh(acc + b_ref[0])

    grid_spec = pltpu.PrefetchScalarGridSpec(
        num_scalar_prefetch=2,
        grid=(NB,),
        in_specs=[
            pl.BlockSpec((R, F), lambda i, be, bfd: (i, 0)),
            pl.BlockSpec((1, F, D), lambda i, be, bfd: (be[i], 0, 0)),
            pl.BlockSpec((1, 1, D), lambda i, be, bfd: (be[i], 0, 0)),
        ],
        out_specs=pl.BlockSpec((R, D), lambda i, be, bfd: (i, 0)),
    )
    return pl.pallas_call(
        body,
        grid_spec=grid_spec,
        out_shape=jax.ShapeDtypeStruct((NPAD, D), jnp.float32),
        compiler_params=pltpu.CompilerParams(dimension_semantics=("arbitrary",)),
    )(block_expert, bfd, x, W.astype(jnp.bfloat16), bias.reshape(E, 1, D))


def _route(bus_type):
    bt = bus_type.astype(jnp.int32)
    ohi = (bt[:, None] == jnp.arange(E, dtype=jnp.int32)[None, :]).astype(
        jnp.int32
    )
    counts = jnp.sum(ohi, axis=0)
    order = jnp.argsort(bt).astype(jnp.int32)
    cum_excl = jnp.cumsum(counts) - counts
    padded = ((counts + R - 1) // R) * R
    ends_p = jnp.cumsum(padded)
    starts_p = ends_p - padded
    slot = jnp.arange(NPAD, dtype=jnp.int32)
    grp = jnp.searchsorted(ends_p, slot, side="right").astype(jnp.int32)
    g = jnp.minimum(grp, E - 1)
    off = slot - starts_p[g]
    src = jnp.clip(cum_excl[g] + jnp.minimum(off, counts[g] - 1), 0, N - 1)
    perm = order[src]
    block_expert = bt[perm[::R]]
    rank = jnp.take_along_axis(
        jnp.cumsum(ohi, axis=0) - ohi, bt[:, None], axis=1
    )[:, 0]
    pos = starts_p[bt] + rank
    return perm, pos, block_expert


def kernel(feat, bus_type, bus_feature_dims, W, bias):
    perm, pos, block_expert = _route(bus_type)
    gathered = _sc_gather(feat, perm.reshape(NW, NG, GCH))
    y = _tc_matmul(
        gathered, W, bias, block_expert, bus_feature_dims.astype(jnp.int32)
    )
    return _sc_unpermute(y, pos.reshape(NW, NSC, SCH))


# speedup vs baseline: 2.6479x; 1.0007x over previous
import functools

import jax
import jax.numpy as jnp
from jax import lax
from jax.experimental import pallas as pl
from jax.experimental.pallas import tpu as pltpu
from jax.experimental.pallas import tpu_sc as plsc

N = 16384
F = 128
E = 8
D = 4096

R = 256
NPAD = N + E * R
NB = NPAD // R
NW = 32
PW = NPAD // NW
GCH = 48
NG = PW // GCH
PW2 = N // NW
SCH = 8
NSC = PW2 // SCH

_MESH = dict(core_axis_name="c", subcore_axis_name="s")


def _sc_gather(feat, perm3):
    @functools.partial(
        pl.kernel,
        mesh=plsc.VectorSubcoreMesh(**_MESH),
        out_type=jax.ShapeDtypeStruct((NPAD, F), jnp.float32),
        scratch_types=[
            pltpu.VMEM((NG, GCH), jnp.int32),
            pltpu.VMEM((PW, F), jnp.float32),
            pltpu.SemaphoreType.DMA,
        ],
    )
    def gk(feat_hbm, idx_hbm, out_hbm, idx_v, rows_v, sem):
        wid = lax.axis_index("s") * 2 + lax.axis_index("c")
        pltpu.sync_copy(idx_hbm.at[wid], idx_v)
        copies = [
            pltpu.async_copy(
                feat_hbm.at[idx_v.at[k]], rows_v.at[pl.ds(k * GCH, GCH)], sem
            )
            for k in range(NG)
        ]
        for c in copies:
            c.wait()
        pltpu.sync_copy(rows_v, out_hbm.at[pl.ds(wid * PW, PW)])

    return gk(feat, perm3)


def _sc_unpermute(y, pos3):
    @functools.partial(
        pl.kernel,
        mesh=plsc.VectorSubcoreMesh(**_MESH),
        out_type=jax.ShapeDtypeStruct((N, D), jnp.float32),
        scratch_types=[
            pltpu.VMEM((NSC, SCH), jnp.int32),
            pltpu.VMEM((3, SCH, D), jnp.float32),
            pltpu.SemaphoreType.DMA,
            pltpu.SemaphoreType.DMA,
        ],
    )
    def uk(y_hbm, idx_hbm, out_hbm, idx_v, buf_v, sem_in, sem_out):
        wid = lax.axis_index("s") * 2 + lax.axis_index("c")
        base = wid * PW2
        pltpu.sync_copy(idx_hbm.at[wid], idx_v)
        reads = [None] * NSC
        writes = [None] * NSC
        for c in range(3):
            reads[c] = pltpu.async_copy(
                y_hbm.at[idx_v.at[c]], buf_v.at[c], sem_in
            )
        for c in range(NSC):
            b = c % 3
            reads[c].wait()
            writes[c] = pltpu.async_copy(
                buf_v.at[b], out_hbm.at[pl.ds(base + c * SCH, SCH)], sem_out
            )
            if c >= 1 and c + 2 < NSC:
                writes[c - 1].wait()
                reads[c + 2] = pltpu.async_copy(
                    y_hbm.at[idx_v.at[c + 2]], buf_v.at[(c + 2) % 3], sem_in
                )
        writes[NSC - 3].wait()
        writes[NSC - 2].wait()
        writes[NSC - 1].wait()

    return uk(y, pos3)


def _tc_matmul(x, W, bias, block_expert, bfd):
    def body(be_ref, bfd_ref, x_ref, w_ref, b_ref, o_ref):
        i = pl.program_id(0)
        end = jnp.minimum(bfd_ref[be_ref[i]], F)
        colmask = (lax.broadcasted_iota(jnp.int32, (1, F), 1) < end).astype(
            jnp.float32
        )
        xm = (x_ref[...] * colmask).astype(jnp.bfloat16)
        acc = jnp.dot(xm, w_ref[0], preferred_element_type=jnp.float32)
        o_ref[...] = jnp.tanh(acc + b_ref[0])

    grid_spec = pltpu.PrefetchScalarGridSpec(
        num_scalar_prefetch=2,
        grid=(NB,),
        in_specs=[
            pl.BlockSpec((R, F), lambda i, be, bfd: (i, 0)),
            pl.BlockSpec((1, F, D), lambda i, be, bfd: (be[i], 0, 0)),
            pl.BlockSpec((1, 1, D), lambda i, be, bfd: (be[i], 0, 0)),
        ],
        out_specs=pl.BlockSpec((R, D), lambda i, be, bfd: (i, 0)),
    )
    return pl.pallas_call(
        body,
        grid_spec=grid_spec,
        out_shape=jax.ShapeDtypeStruct((NPAD, D), jnp.float32),
        compiler_params=pltpu.CompilerParams(dimension_semantics=("arbitrary",)),
    )(block_expert, bfd, x, W.astype(jnp.bfloat16), bias.reshape(E, 1, D))


def _route(bus_type):
    bt = bus_type.astype(jnp.int32)
    ohi = (bt[:, None] == jnp.arange(E, dtype=jnp.int32)[None, :]).astype(
        jnp.int32
    )
    counts = jnp.sum(ohi, axis=0)
    order = jnp.argsort(bt).astype(jnp.int32)
    cum_excl = jnp.cumsum(counts) - counts
    padded = ((counts + R - 1) // R) * R
    ends_p = jnp.cumsum(padded)
    starts_p = ends_p - padded
    slot = jnp.arange(NPAD, dtype=jnp.int32)
    grp = jnp.searchsorted(ends_p, slot, side="right").astype(jnp.int32)
    g = jnp.minimum(grp, E - 1)
    off = slot - starts_p[g]
    src = jnp.clip(cum_excl[g] + jnp.minimum(off, counts[g] - 1), 0, N - 1)
    perm = order[src]
    block_expert = bt[perm[::R]]
    rank = jnp.take_along_axis(
        jnp.cumsum(ohi, axis=0) - ohi, bt[:, None], axis=1
    )[:, 0]
    pos = starts_p[bt] + rank
    return perm, pos, block_expert


def kernel(feat, bus_type, bus_feature_dims, W, bias):
    perm, pos, block_expert = _route(bus_type)
    gathered = _sc_gather(feat, perm.reshape(NW, NG, GCH))
    y = _tc_matmul(
        gathered, W, bias, block_expert, bus_feature_dims.astype(jnp.int32)
    )
    return _sc_unpermute(y, pos.reshape(NW, NSC, SCH))


# speedup vs baseline: 2.8481x; 1.0756x over previous
import functools

import jax
import jax.numpy as jnp
from jax import lax
from jax.experimental import pallas as pl
from jax.experimental.pallas import tpu as pltpu
from jax.experimental.pallas import tpu_sc as plsc

N = 16384
F = 128
E = 8
D = 4096

R = 256
NPAD = N + E * R
NB = NPAD // R
NW = 32
PW = NPAD // NW
GCH = 48
NG = PW // GCH
PW2 = N // NW
SCH = 8
NSC = PW2 // SCH

_MESH = dict(core_axis_name="c", subcore_axis_name="s")


def _sc_gather(feat, perm3):
    @functools.partial(
        pl.kernel,
        mesh=plsc.VectorSubcoreMesh(**_MESH),
        out_type=jax.ShapeDtypeStruct((NPAD, F), jnp.float32),
        scratch_types=[
            pltpu.VMEM((NG, GCH), jnp.int32),
            pltpu.VMEM((PW, F), jnp.float32),
            pltpu.SemaphoreType.DMA,
        ],
    )
    def gk(feat_hbm, idx_hbm, out_hbm, idx_v, rows_v, sem):
        wid = lax.axis_index("s") * 2 + lax.axis_index("c")
        pltpu.sync_copy(idx_hbm.at[wid], idx_v)
        copies = [
            pltpu.async_copy(
                feat_hbm.at[idx_v.at[k]], rows_v.at[pl.ds(k * GCH, GCH)], sem
            )
            for k in range(NG)
        ]
        for c in copies:
            c.wait()
        pltpu.sync_copy(rows_v, out_hbm.at[pl.ds(wid * PW, PW)])

    return gk(feat, perm3)


def _sc_unpermute(y, pos3):
    @functools.partial(
        pl.kernel,
        mesh=plsc.VectorSubcoreMesh(**_MESH),
        out_type=jax.ShapeDtypeStruct((N, D), jnp.float32),
        scratch_types=[
            pltpu.VMEM((NSC, SCH), jnp.int32),
            pltpu.VMEM((3, SCH, D), jnp.float32),
            pltpu.SemaphoreType.DMA,
            pltpu.SemaphoreType.DMA,
        ],
    )
    def uk(y_hbm, idx_hbm, out_hbm, idx_v, buf_v, sem_in, sem_out):
        wid = lax.axis_index("s") * 2 + lax.axis_index("c")
        base = wid * PW2
        pltpu.sync_copy(idx_hbm.at[wid], idx_v)
        reads = [None] * NSC
        writes = [None] * NSC
        for c in range(3):
            reads[c] = pltpu.async_copy(
                y_hbm.at[idx_v.at[c]], buf_v.at[c], sem_in
            )
        for c in range(NSC):
            b = c % 3
            reads[c].wait()
            writes[c] = pltpu.async_copy(
                buf_v.at[b], out_hbm.at[pl.ds(base + c * SCH, SCH)], sem_out
            )
            if c >= 1 and c + 2 < NSC:
                writes[c - 1].wait()
                reads[c + 2] = pltpu.async_copy(
                    y_hbm.at[idx_v.at[c + 2]], buf_v.at[(c + 2) % 3], sem_in
                )
        writes[NSC - 3].wait()
        writes[NSC - 2].wait()
        writes[NSC - 1].wait()

    return uk(y, pos3)


def _tc_matmul(x, W, bias, block_expert, bfd):
    def body(be_ref, bfd_ref, x_ref, w_ref, b_ref, o_ref):
        i = pl.program_id(0)
        end = jnp.minimum(bfd_ref[be_ref[i]], F)
        colmask = (lax.broadcasted_iota(jnp.int32, (1, F), 1) < end).astype(
            jnp.float32
        )
        xm = (x_ref[...] * colmask).astype(jnp.bfloat16)
        acc = jnp.dot(xm, w_ref[0], preferred_element_type=jnp.float32)
        o_ref[...] = jnp.tanh(acc + b_ref[0])

    grid_spec = pltpu.PrefetchScalarGridSpec(
        num_scalar_prefetch=2,
        grid=(NB,),
        in_specs=[
            pl.BlockSpec((R, F), lambda i, be, bfd: (i, 0)),
            pl.BlockSpec((1, F, D), lambda i, be, bfd: (be[i], 0, 0)),
            pl.BlockSpec((1, 1, D), lambda i, be, bfd: (be[i], 0, 0)),
        ],
        out_specs=pl.BlockSpec((R, D), lambda i, be, bfd: (i, 0)),
    )
    return pl.pallas_call(
        body,
        grid_spec=grid_spec,
        out_shape=jax.ShapeDtypeStruct((NPAD, D), jnp.float32),
        compiler_params=pltpu.CompilerParams(dimension_semantics=("arbitrary",)),
    )(block_expert, bfd, x, W.astype(jnp.bfloat16), bias.reshape(E, 1, D))


def _route(bus_type):
    bt = bus_type.astype(jnp.int32)
    counts = jnp.sum(
        (bt[:, None] == jnp.arange(E, dtype=jnp.int32)[None, :]).astype(
            jnp.int32
        ),
        axis=0,
    )
    order = jnp.argsort(bt).astype(jnp.int32)
    cum_excl = jnp.cumsum(counts) - counts
    padded = ((counts + R - 1) // R) * R
    ends_p = jnp.cumsum(padded)
    starts_p = ends_p - padded
    bstart = jnp.arange(NB, dtype=jnp.int32) * R
    bg = jnp.minimum(
        jnp.sum((bstart[:, None] >= ends_p[None, :]).astype(jnp.int32), axis=1),
        E - 1,
    )
    g = jnp.broadcast_to(bg[:, None], (NB, R)).reshape(NPAD)
    off = jnp.arange(NPAD, dtype=jnp.int32) - starts_p[g]
    src = jnp.clip(cum_excl[g] + jnp.minimum(off, counts[g] - 1), 0, N - 1)
    perm = order[src]
    block_expert = bt[perm[::R]]
    inv = jnp.zeros((N,), jnp.int32).at[order].set(
        jnp.arange(N, dtype=jnp.int32)
    )
    pos = starts_p[bt] + inv - cum_excl[bt]
    return perm, pos, block_expert


def kernel(feat, bus_type, bus_feature_dims, W, bias):
    perm, pos, block_expert = _route(bus_type)
    gathered = _sc_gather(feat, perm.reshape(NW, NG, GCH))
    y = _tc_matmul(
        gathered, W, bias, block_expert, bus_feature_dims.astype(jnp.int32)
    )
    return _sc_unpermute(y, pos.reshape(NW, NSC, SCH))
